# baseline (speedup 1.0000x reference)
import functools

import jax
import jax.numpy as jnp
from jax import lax
from jax.experimental import pallas as pl
from jax.experimental.pallas import tpu as pltpu
from jax.experimental.pallas import tpu_sc as plsc

_B = 8
_N = 2048
_ROWS_TOTAL = _B * _N
_NW = 32
_PER_W = _ROWS_TOTAL // _NW
_RB = 8
_CW = _RB * _N
_NBUF = 7
_CHUNKS = _PER_W // _RB

_mesh = plsc.VectorSubcoreMesh(core_axis_name="c", subcore_axis_name="s")


@functools.partial(
    pl.kernel,
    mesh=_mesh,
    out_type=jax.ShapeDtypeStruct((_ROWS_TOTAL * _N,), jnp.float32),
    scratch_types=[pltpu.VMEM((_CW,), jnp.float32)] * _NBUF
    + [
        pltpu.VMEM((_PER_W,), jnp.float32),
        pltpu.VMEM((16,), jnp.float32),
    ]
    + [pltpu.SemaphoreType.DMA] * (2 * _NBUF),
)
def _sc_body(cov_hbm, emb_hbm, ns_hbm, out_hbm, *rest):
    bufs = rest[:_NBUF]
    emb_v, ns_v = rest[_NBUF], rest[_NBUF + 1]
    in_sems = rest[_NBUF + 2 : _NBUF + 2 + _NBUF]
    out_sems = rest[_NBUF + 2 + _NBUF :]
    wid = lax.axis_index("s") * 2 + lax.axis_index("c")
    base = wid * _PER_W

    pltpu.sync_copy(emb_hbm.at[pl.ds(base, _PER_W)], emb_v)
    pltpu.sync_copy(ns_hbm, ns_v)
    ns = jnp.exp(ns_v[...])
    lane = lax.iota(jnp.int32, 16)
    col0 = lax.rem(base, _N)

    def in_copy(k):
        return pltpu.make_async_copy(
            cov_hbm.at[pl.ds((base + k * _RB) * _N, _CW)],
            bufs[k % _NBUF],
            in_sems[k % _NBUF],
        )

    def out_copy(k):
        return pltpu.make_async_copy(
            bufs[k % _NBUF],
            out_hbm.at[pl.ds((base + k * _RB) * _N, _CW)],
            out_sems[k % _NBUF],
        )

    for j in range(min(_NBUF, _CHUNKS)):
        in_copy(j).start()

    waited_out = set()
    for k in range(_CHUNKS):
        in_copy(k).wait()
        s = (k * _RB) % 16
        val = jnp.exp(emb_v[pl.ds((k * _RB // 16) * 16, 16)]) + ns
        buf = bufs[k % _NBUF]

        def _fix_row(rr, _, buf=buf, val=val, s=s, base=col0 + (k * _RB // 16) * 16):
            off = rr * _N + base
            buf[pl.ds(off, 16)] = buf[pl.ds(off, 16)] + jnp.where(
                lane == rr + s, val, 0.0
            )
            return 0

        lax.fori_loop(0, _RB, _fix_row, 0)
        out_copy(k).start()
        j = k - 3
        if j >= 0 and j + _NBUF < _CHUNKS:
            out_copy(j).wait()
            waited_out.add(j)
            in_copy(j + _NBUF).start()
    for k in range(_CHUNKS):
        if k not in waited_out:
            out_copy(k).wait()


def kernel(cov, embeddings, noise_scale):
    cov1d = cov.reshape(_ROWS_TOTAL * _N)
    emb = embeddings[:, :, -1].reshape(_ROWS_TOTAL)
    ns16 = jnp.broadcast_to(noise_scale, (16,))
    out = _sc_body(cov1d, emb, ns16)
    return out.reshape(_B, _N, _N)


# speedup vs baseline: 3.7717x; 3.7717x over previous
import jax
import jax.numpy as jnp
from jax.experimental import pallas as pl
from jax.experimental.pallas import tpu as pltpu

_B = 8
_N = 2048
_NBUF = 3


def _body(emb_ref, ns_ref, cov_hbm, out_hbm, buf, in_sems, out_sems):
    def in_copy(k):
        return pltpu.make_async_copy(
            cov_hbm.at[k], buf.at[k % _NBUF], in_sems.at[k % _NBUF]
        )

    def out_copy(k):
        return pltpu.make_async_copy(
            buf.at[k % _NBUF], out_hbm.at[k], out_sems.at[k % _NBUF]
        )

    row = jax.lax.broadcasted_iota(jnp.int32, (_N, _N), 0)
    col = jax.lax.broadcasted_iota(jnp.int32, (_N, _N), 1)
    mask = row == col
    ns = jnp.exp(ns_ref[0, 0])

    for j in range(_NBUF):
        in_copy(j).start()

    waited_out = set()
    for k in range(_B):
        in_copy(k).wait()
        ev = jnp.exp(emb_ref[k]) + ns
        i = k % _NBUF
        buf[i] = buf[i] + jnp.where(mask, ev, 0.0)
        out_copy(k).start()
        j = k - 2
        if j >= 0 and j + _NBUF < _B:
            out_copy(j).wait()
            waited_out.add(j)
            in_copy(j + _NBUF).start()
    for k in range(_B):
        if k not in waited_out:
            out_copy(k).wait()


def kernel(cov, embeddings, noise_scale):
    emb = embeddings[:, :, -1].reshape(_B, 1, _N)
    ns = noise_scale.reshape(1, 1)
    return pl.pallas_call(
        _body,
        in_specs=[
            pl.BlockSpec(memory_space=pltpu.MemorySpace.VMEM),
            pl.BlockSpec(memory_space=pltpu.MemorySpace.VMEM),
            pl.BlockSpec(memory_space=pl.ANY),
        ],
        out_specs=pl.BlockSpec(memory_space=pl.ANY),
        out_shape=jax.ShapeDtypeStruct((_B, _N, _N), jnp.float32),
        scratch_shapes=[
            pltpu.VMEM((_NBUF, _N, _N), jnp.float32),
            pltpu.SemaphoreType.DMA((_NBUF,)),
            pltpu.SemaphoreType.DMA((_NBUF,)),
        ],
    )(emb, ns, cov)


# speedup vs baseline: 4.0763x; 1.0808x over previous
import jax
import jax.numpy as jnp
from jax.experimental import pallas as pl
from jax.experimental.pallas import tpu as pltpu

_B = 8
_N = 2048
_ROWS = 1024


def _stripe_kernel(emb_ref, ns_ref, cov_ref, out_ref):
    i = pl.program_id(1)
    out_ref[0] = cov_ref[0]
    ev = jnp.exp(emb_ref[0, :, pl.ds(i * _ROWS, _ROWS)]) + jnp.exp(ns_ref[0, 0])
    row = jax.lax.broadcasted_iota(jnp.int32, (_ROWS, _ROWS), 0)
    col = jax.lax.broadcasted_iota(jnp.int32, (_ROWS, _ROWS), 1)
    sub = out_ref[0, :, pl.ds(i * _ROWS, _ROWS)]
    out_ref[0, :, pl.ds(i * _ROWS, _ROWS)] = sub + jnp.where(row == col, ev, 0.0)


def kernel(cov, embeddings, noise_scale):
    emb = embeddings[:, :, -1].reshape(_B, 1, _N)
    ns = noise_scale.reshape(1, 1)
    return pl.pallas_call(
        _stripe_kernel,
        grid=(_B, _N // _ROWS),
        in_specs=[
            pl.BlockSpec((1, 1, _N), lambda b, i: (b, 0, 0)),
            pl.BlockSpec((1, 1), lambda b, i: (0, 0)),
            pl.BlockSpec((1, _ROWS, _N), lambda b, i: (b, i, 0)),
        ],
        out_specs=pl.BlockSpec((1, _ROWS, _N), lambda b, i: (b, i, 0)),
        out_shape=jax.ShapeDtypeStruct((_B, _N, _N), jnp.float32),
        compiler_params=pltpu.CompilerParams(dimension_semantics=("parallel", "parallel")),
    )(emb, ns, cov)
